# baseline (speedup 1.0000x reference)
import jax
import jax.numpy as jnp
from jax import lax
from jax.experimental import pallas as pl
from jax.experimental.pallas import tpu as pltpu
from jax.experimental.pallas import tpu_sc as plsc

_LENGTH = 8192
_DIM = 16
_NUM_CORES = 2
_NUM_SUBCORES = 16
_NUM_WORKERS = _NUM_CORES * _NUM_SUBCORES
_COLS = _LENGTH // _NUM_WORKERS
_ALIGN = 128
_WIN = _COLS + _ALIGN
_LANES = 16


def _slice_copy(pos_hbm, embt_hbm, out_hbm, pos_v, buf_v, buf2_v, sem_a, sem_b, sem_o):
    wid = lax.axis_index("s") * _NUM_CORES + lax.axis_index("c")
    base = wid * _COLS
    half = _COLS // 2
    pltpu.sync_copy(pos_hbm, pos_v.at[pl.ds(0, 1)])
    p = pos_v[...][0]
    r = lax.rem(p, _ALIGN)
    astart = pl.multiple_of(p - r + base, _ALIGN)
    cp_a = pltpu.async_copy(
        embt_hbm.at[:, pl.ds(astart, _COLS)], buf_v.at[:, pl.ds(0, _COLS)], sem_a
    )
    cp_b = pltpu.async_copy(
        embt_hbm.at[:, pl.ds(astart + _COLS, _ALIGN)],
        buf_v.at[:, pl.ds(_COLS, _ALIGN)],
        sem_b,
    )
    lanes = jax.lax.iota(jnp.int32, _LANES)
    rvec = jnp.full((_LANES,), r, jnp.int32) + lanes

    def _shift(j_lo, j_hi):
        def body(s, carry):
            row = jnp.full((_LANES,), s, jnp.int32)
            for j in range(j_lo, j_hi):
                v = plsc.load_gather(buf_v, [row, rvec + (j * _LANES)])
                buf2_v[s, pl.ds(j * _LANES, _LANES)] = v
            return carry

        lax.fori_loop(0, _DIM, body, 0)

    nj = _COLS // _LANES
    cp_a.wait()
    _shift(0, nj // 2)
    cp_o = pltpu.async_copy(
        buf2_v.at[:, pl.ds(0, half)], out_hbm.at[:, pl.ds(base, half)], sem_o
    )
    cp_b.wait()
    _shift(nj // 2, nj)
    cp_o.wait()
    pltpu.sync_copy(
        buf2_v.at[:, pl.ds(half, half)], out_hbm.at[:, pl.ds(base + half, half)]
    )


def kernel(pos, embed):
    pos32 = pos.astype(jnp.int32)
    embed_t = embed.T
    mesh = plsc.VectorSubcoreMesh(core_axis_name="c", subcore_axis_name="s")
    run = pl.kernel(
        _slice_copy,
        mesh=mesh,
        out_type=jax.ShapeDtypeStruct((_DIM, _LENGTH), jnp.float32),
        scratch_types=[
            pltpu.VMEM((16,), jnp.int32),
            pltpu.VMEM((_DIM, _WIN), jnp.float32),
            pltpu.VMEM((_DIM, _COLS), jnp.float32),
            pltpu.SemaphoreType.DMA,
            pltpu.SemaphoreType.DMA,
            pltpu.SemaphoreType.DMA,
        ],
        compiler_params=pltpu.CompilerParams(needs_layout_passes=False),
    )
    return run(pos32, embed_t).T
